# Initial kernel scaffold; baseline (speedup 1.0000x reference)
import jax
import jax.numpy as jnp
from jax.experimental import pallas as pl


def kernel(x, init_state, init_val, route_U, route_K, vn_gamma, vn_beta, read_W, read_gate):
    raise NotImplementedError("write your pallas kernel here")


# speedup vs baseline: 13.4141x; 13.4141x over previous
import functools

import jax
import jax.numpy as jnp
from jax import lax
from jax.experimental import pallas as pl

B, T, D, S, R, K = 1, 2048, 1024, 8192, 64, 32

TBLK_A = 256
SBLK_B = 1024
TBLK_D = 64


def _layernorm_rows(v, g, b, eps=1e-5):
    m = jnp.mean(v, axis=-1, keepdims=True)
    var = jnp.mean((v - m) * (v - m), axis=-1, keepdims=True)
    return (v - m) / jnp.sqrt(var + eps) * g + b


def _topk_mask(a, sentinel):
    def body(_, A):
        m = jnp.max(A, axis=1, keepdims=True)
        return jnp.where(A == m, sentinel, A)
    A = lax.fori_loop(0, K, body, a, unroll=True)
    return A == sentinel


def _kA(x_ref, U_ref, Kt_ref, rW_ref, W_ref, q_ref):
    x = x_ref[...]
    xu = jnp.dot(x, U_ref[...], preferred_element_type=jnp.float32)
    s = lax.dot_general(xu, Kt_ref[...], (((1,), (1,)), ((), ())),
                        preferred_element_type=jnp.float32)
    a = jnp.abs(s)
    m1 = jnp.max(a, axis=1, keepdims=True)
    sel = _topk_mask(a, -1.0)
    e = jnp.where(sel, jnp.exp(a - m1), 0.0)
    Z = jnp.sum(e, axis=1, keepdims=True)
    W_ref[...] = jnp.sign(s) * (e / Z)
    q_ref[...] = jnp.dot(x, rW_ref[...], preferred_element_type=jnp.float32)


def _kB(W_ref, x_ref, iv_ref, g_ref, b_ref, out_ref):
    dval = lax.dot_general(W_ref[...], x_ref[...], (((0,), (0,)), ((), ())),
                           preferred_element_type=jnp.float32)
    g = g_ref[...]
    b = b_ref[...]
    v0 = _layernorm_rows(iv_ref[...], g, b)
    out_ref[...] = _layernorm_rows(v0 + dval, g, b)


def _kD(q_ref, x_ref, V_ref, gate_ref, y_ref):
    q = q_ref[...]
    V = V_ref[...]
    rs = lax.dot_general(q, V, (((1,), (1,)), ((), ())),
                         preferred_element_type=jnp.float32)
    m1 = jnp.max(rs, axis=1, keepdims=True)
    sel = _topk_mask(rs, -jnp.inf)
    e = jnp.where(sel, jnp.exp(rs - m1), 0.0)
    Z = jnp.sum(e, axis=1, keepdims=True)
    ro = jnp.dot(e / Z, V, preferred_element_type=jnp.float32)
    y_ref[...] = x_ref[...] + gate_ref[0, 0] * ro


@jax.jit
def kernel(x, init_state, init_val, route_U, route_K, vn_gamma, vn_beta,
           read_W, read_gate):
    del init_state
    x2 = x.reshape(T, D)
    gamma = vn_gamma.reshape(1, D)
    beta = vn_beta.reshape(1, D)
    gate = read_gate.reshape(1, 1)

    W, q = pl.pallas_call(
        _kA,
        grid=(T // TBLK_A,),
        in_specs=[
            pl.BlockSpec((TBLK_A, D), lambda t: (t, 0)),
            pl.BlockSpec((D, R), lambda t: (0, 0)),
            pl.BlockSpec((S, R), lambda t: (0, 0)),
            pl.BlockSpec((D, D), lambda t: (0, 0)),
        ],
        out_specs=[
            pl.BlockSpec((TBLK_A, S), lambda t: (t, 0)),
            pl.BlockSpec((TBLK_A, D), lambda t: (t, 0)),
        ],
        out_shape=[
            jax.ShapeDtypeStruct((T, S), jnp.float32),
            jax.ShapeDtypeStruct((T, D), jnp.float32),
        ],
    )(x2, route_U, route_K, read_W)

    val_new = pl.pallas_call(
        _kB,
        grid=(S // SBLK_B,),
        in_specs=[
            pl.BlockSpec((T, SBLK_B), lambda s: (0, s)),
            pl.BlockSpec((T, D), lambda s: (0, 0)),
            pl.BlockSpec((SBLK_B, D), lambda s: (s, 0)),
            pl.BlockSpec((1, D), lambda s: (0, 0)),
            pl.BlockSpec((1, D), lambda s: (0, 0)),
        ],
        out_specs=pl.BlockSpec((SBLK_B, D), lambda s: (s, 0)),
        out_shape=jax.ShapeDtypeStruct((S, D), jnp.float32),
    )(W, x2, init_val, gamma, beta)

    y = pl.pallas_call(
        _kD,
        grid=(T // TBLK_D,),
        in_specs=[
            pl.BlockSpec((TBLK_D, D), lambda t: (t, 0)),
            pl.BlockSpec((TBLK_D, D), lambda t: (t, 0)),
            pl.BlockSpec((S, D), lambda t: (0, 0)),
            pl.BlockSpec((1, 1), lambda t: (0, 0)),
        ],
        out_specs=pl.BlockSpec((TBLK_D, D), lambda t: (t, 0)),
        out_shape=jax.ShapeDtypeStruct((T, D), jnp.float32),
    )(q, x2, val_new, gate)

    return y.reshape(B, T, D)


# speedup vs baseline: 19.3431x; 1.4420x over previous
import jax
import jax.numpy as jnp
from jax import lax
from jax.experimental import pallas as pl

B, T, D, S, R, K = 1, 2048, 1024, 8192, 64, 32

TBLK_A = 256
SBLK_B = 1024
TBLK_D1 = 128
TBLK_D2 = 256
NROUNDS = 20


def _layernorm_rows(v, g, b, eps=1e-5):
    m = jnp.mean(v, axis=-1, keepdims=True)
    var = jnp.mean((v - m) * (v - m), axis=-1, keepdims=True)
    return (v - m) / jnp.sqrt(var + eps) * g + b


def _topk_threshold(a, lo, hi):
    kf = jnp.float32(K)
    for _ in range(NROUNDS):
        mid = 0.5 * (lo + hi)
        cnt = jnp.sum((a >= mid).astype(jnp.float32), axis=1, keepdims=True)
        ge = cnt >= kf
        lo = jnp.where(ge, mid, lo)
        hi = jnp.where(ge, hi, mid)
    return lo


def _split_bf16(v):
    hi = v.astype(jnp.bfloat16)
    lo = (v - hi.astype(jnp.float32)).astype(jnp.bfloat16)
    return hi, lo


def _kA(x_ref, U_ref, Kt_ref, rW_ref, W_ref, qh_ref, ql_ref):
    x = x_ref[...]
    xu = jnp.dot(x, U_ref[...], preferred_element_type=jnp.float32)
    s = lax.dot_general(xu, Kt_ref[...], (((1,), (1,)), ((), ())),
                        preferred_element_type=jnp.float32)
    a = jnp.abs(s)
    m1 = jnp.max(a, axis=1, keepdims=True)
    thr = _topk_threshold(a, jnp.zeros_like(m1), m1)
    e = jnp.where(a >= thr, jnp.exp(a - m1), 0.0)
    Z = jnp.sum(e, axis=1, keepdims=True)
    W_ref[...] = (jnp.sign(s) * (e / Z)).astype(jnp.bfloat16)
    q = jnp.dot(x, rW_ref[...], preferred_element_type=jnp.float32)
    qh, ql = _split_bf16(q)
    qh_ref[...] = qh
    ql_ref[...] = ql


def _kB(W_ref, x_ref, iv_ref, g_ref, b_ref, vh_ref, vth_ref, vtl_ref):
    dval = lax.dot_general(W_ref[...], x_ref[...], (((0,), (0,)), ((), ())),
                           preferred_element_type=jnp.float32)
    g = g_ref[...]
    b = b_ref[...]
    v0 = _layernorm_rows(iv_ref[...], g, b)
    v = _layernorm_rows(v0 + dval, g, b)
    vh_ref[...] = v.astype(jnp.bfloat16)
    vt = v.T
    vth, vtl = _split_bf16(vt)
    vth_ref[...] = vth
    vtl_ref[...] = vtl


def _kD1(qh_ref, ql_ref, VTh_ref, VTl_ref, W2_ref):
    qh = qh_ref[...]
    VTh = VTh_ref[...]
    rs = (jnp.dot(qh, VTh, preferred_element_type=jnp.float32)
          + jnp.dot(qh, VTl_ref[...], preferred_element_type=jnp.float32)
          + jnp.dot(ql_ref[...], VTh, preferred_element_type=jnp.float32))
    m1 = jnp.max(rs, axis=1, keepdims=True)
    m0 = jnp.min(rs, axis=1, keepdims=True)
    thr = _topk_threshold(rs, m0, m1)
    e = jnp.where(rs >= thr, jnp.exp(rs - m1), 0.0)
    Z = jnp.sum(e, axis=1, keepdims=True)
    W2_ref[...] = (e / Z).astype(jnp.bfloat16)


def _kD2(W2_ref, Vh_ref, x_ref, gate_ref, y_ref):
    ro = jnp.dot(W2_ref[...], Vh_ref[...], preferred_element_type=jnp.float32)
    y_ref[...] = x_ref[...] + gate_ref[0, 0] * ro


@jax.jit
def kernel(x, init_state, init_val, route_U, route_K, vn_gamma, vn_beta,
           read_W, read_gate):
    del init_state
    x2 = x.reshape(T, D)
    gamma = vn_gamma.reshape(1, D)
    beta = vn_beta.reshape(1, D)
    gate = read_gate.reshape(1, 1)

    W, qh, ql = pl.pallas_call(
        _kA,
        grid=(T // TBLK_A,),
        in_specs=[
            pl.BlockSpec((TBLK_A, D), lambda t: (t, 0)),
            pl.BlockSpec((D, R), lambda t: (0, 0)),
            pl.BlockSpec((S, R), lambda t: (0, 0)),
            pl.BlockSpec((D, D), lambda t: (0, 0)),
        ],
        out_specs=[
            pl.BlockSpec((TBLK_A, S), lambda t: (t, 0)),
            pl.BlockSpec((TBLK_A, D), lambda t: (t, 0)),
            pl.BlockSpec((TBLK_A, D), lambda t: (t, 0)),
        ],
        out_shape=[
            jax.ShapeDtypeStruct((T, S), jnp.bfloat16),
            jax.ShapeDtypeStruct((T, D), jnp.bfloat16),
            jax.ShapeDtypeStruct((T, D), jnp.bfloat16),
        ],
    )(x2, route_U, route_K, read_W)

    vh, vth, vtl = pl.pallas_call(
        _kB,
        grid=(S // SBLK_B,),
        in_specs=[
            pl.BlockSpec((T, SBLK_B), lambda s: (0, s)),
            pl.BlockSpec((T, D), lambda s: (0, 0)),
            pl.BlockSpec((SBLK_B, D), lambda s: (s, 0)),
            pl.BlockSpec((1, D), lambda s: (0, 0)),
            pl.BlockSpec((1, D), lambda s: (0, 0)),
        ],
        out_specs=[
            pl.BlockSpec((SBLK_B, D), lambda s: (s, 0)),
            pl.BlockSpec((D, SBLK_B), lambda s: (0, s)),
            pl.BlockSpec((D, SBLK_B), lambda s: (0, s)),
        ],
        out_shape=[
            jax.ShapeDtypeStruct((S, D), jnp.bfloat16),
            jax.ShapeDtypeStruct((D, S), jnp.bfloat16),
            jax.ShapeDtypeStruct((D, S), jnp.bfloat16),
        ],
    )(W, x2.astype(jnp.bfloat16), init_val, gamma, beta)

    W2 = pl.pallas_call(
        _kD1,
        grid=(T // TBLK_D1,),
        in_specs=[
            pl.BlockSpec((TBLK_D1, D), lambda t: (t, 0)),
            pl.BlockSpec((TBLK_D1, D), lambda t: (t, 0)),
            pl.BlockSpec((D, S), lambda t: (0, 0)),
            pl.BlockSpec((D, S), lambda t: (0, 0)),
        ],
        out_specs=pl.BlockSpec((TBLK_D1, S), lambda t: (t, 0)),
        out_shape=jax.ShapeDtypeStruct((T, S), jnp.bfloat16),
    )(qh, ql, vth, vtl)

    y = pl.pallas_call(
        _kD2,
        grid=(T // TBLK_D2,),
        in_specs=[
            pl.BlockSpec((TBLK_D2, S), lambda t: (t, 0)),
            pl.BlockSpec((S, D), lambda t: (0, 0)),
            pl.BlockSpec((TBLK_D2, D), lambda t: (t, 0)),
            pl.BlockSpec((1, 1), lambda t: (0, 0)),
        ],
        out_specs=pl.BlockSpec((TBLK_D2, D), lambda t: (t, 0)),
        out_shape=jax.ShapeDtypeStruct((T, D), jnp.float32),
    )(W2, vh, x2, gate)

    return y.reshape(B, T, D)


# speedup vs baseline: 24.8122x; 1.2827x over previous
import jax
import jax.numpy as jnp
from jax import lax
from jax.experimental import pallas as pl

B, T, D, S, R, K = 1, 2048, 1024, 8192, 64, 32

TBLK_A = 256
SBLK_B = 1024
TBLK_D1 = 128
TBLK_D2 = 256
NROUNDS = 10


def _layernorm_rows(v, g, b, eps=1e-5):
    m = jnp.mean(v, axis=-1, keepdims=True)
    var = jnp.mean((v - m) * (v - m), axis=-1, keepdims=True)
    return (v - m) / jnp.sqrt(var + eps) * g + b


def _topk_threshold(a, lo, hi):
    kf = jnp.float32(K)
    for _ in range(NROUNDS):
        mid = 0.5 * (lo + hi)
        cnt = jnp.sum((a >= mid).astype(jnp.float32), axis=1, keepdims=True)
        ge = cnt >= kf
        lo = jnp.where(ge, mid, lo)
        hi = jnp.where(ge, hi, mid)
    return lo


def _split_bf16(v):
    hi = v.astype(jnp.bfloat16)
    lo = (v - hi.astype(jnp.float32)).astype(jnp.bfloat16)
    return hi, lo


def _kA(x_ref, U_ref, Kt_ref, rW_ref, W_ref, qh_ref, ql_ref):
    x = x_ref[...]
    xu = jnp.dot(x, U_ref[...], preferred_element_type=jnp.float32)
    s = lax.dot_general(xu, Kt_ref[...], (((1,), (1,)), ((), ())),
                        preferred_element_type=jnp.float32)
    a = jnp.abs(s)
    m1 = jnp.max(a, axis=1, keepdims=True)
    thr = _topk_threshold(a, jnp.zeros_like(m1), m1)
    e = jnp.where(a >= thr, jnp.exp(a - m1), 0.0)
    Z = jnp.sum(e, axis=1, keepdims=True)
    W_ref[...] = (jnp.sign(s) * (e / Z)).astype(jnp.bfloat16)
    q = jnp.dot(x, rW_ref[...], preferred_element_type=jnp.float32)
    qh, ql = _split_bf16(q)
    qh_ref[...] = qh
    ql_ref[...] = ql


def _kB(W_ref, x_ref, iv_ref, g_ref, b_ref, vh_ref, vth_ref, vtl_ref):
    dval = lax.dot_general(W_ref[...], x_ref[...], (((0,), (0,)), ((), ())),
                           preferred_element_type=jnp.float32)
    g = g_ref[...]
    b = b_ref[...]
    v0 = _layernorm_rows(iv_ref[...], g, b)
    v = _layernorm_rows(v0 + dval, g, b)
    vh_ref[...] = v.astype(jnp.bfloat16)
    vt = v.T
    vth, vtl = _split_bf16(vt)
    vth_ref[...] = vth
    vtl_ref[...] = vtl


def _kD1(qh_ref, ql_ref, VTh_ref, VTl_ref, W2_ref):
    qh = qh_ref[...]
    VTh = VTh_ref[...]
    rs = (jnp.dot(qh, VTh, preferred_element_type=jnp.float32)
          + jnp.dot(qh, VTl_ref[...], preferred_element_type=jnp.float32)
          + jnp.dot(ql_ref[...], VTh, preferred_element_type=jnp.float32))
    m1 = jnp.max(rs, axis=1, keepdims=True)
    m0 = jnp.min(rs, axis=1, keepdims=True)
    thr = _topk_threshold(rs, m0, m1)
    e = jnp.where(rs >= thr, jnp.exp(rs - m1), 0.0)
    Z = jnp.sum(e, axis=1, keepdims=True)
    W2_ref[...] = (e / Z).astype(jnp.bfloat16)


def _kD2(W2_ref, Vh_ref, x_ref, gate_ref, y_ref):
    ro = jnp.dot(W2_ref[...], Vh_ref[...], preferred_element_type=jnp.float32)
    y_ref[...] = x_ref[...] + gate_ref[0, 0] * ro


@jax.jit
def kernel(x, init_state, init_val, route_U, route_K, vn_gamma, vn_beta,
           read_W, read_gate):
    del init_state
    x2 = x.reshape(T, D)
    gamma = vn_gamma.reshape(1, D)
    beta = vn_beta.reshape(1, D)
    gate = read_gate.reshape(1, 1)

    W, qh, ql = pl.pallas_call(
        _kA,
        grid=(T // TBLK_A,),
        in_specs=[
            pl.BlockSpec((TBLK_A, D), lambda t: (t, 0)),
            pl.BlockSpec((D, R), lambda t: (0, 0)),
            pl.BlockSpec((S, R), lambda t: (0, 0)),
            pl.BlockSpec((D, D), lambda t: (0, 0)),
        ],
        out_specs=[
            pl.BlockSpec((TBLK_A, S), lambda t: (t, 0)),
            pl.BlockSpec((TBLK_A, D), lambda t: (t, 0)),
            pl.BlockSpec((TBLK_A, D), lambda t: (t, 0)),
        ],
        out_shape=[
            jax.ShapeDtypeStruct((T, S), jnp.bfloat16),
            jax.ShapeDtypeStruct((T, D), jnp.bfloat16),
            jax.ShapeDtypeStruct((T, D), jnp.bfloat16),
        ],
    )(x2, route_U, route_K, read_W)

    vh, vth, vtl = pl.pallas_call(
        _kB,
        grid=(S // SBLK_B,),
        in_specs=[
            pl.BlockSpec((T, SBLK_B), lambda s: (0, s)),
            pl.BlockSpec((T, D), lambda s: (0, 0)),
            pl.BlockSpec((SBLK_B, D), lambda s: (s, 0)),
            pl.BlockSpec((1, D), lambda s: (0, 0)),
            pl.BlockSpec((1, D), lambda s: (0, 0)),
        ],
        out_specs=[
            pl.BlockSpec((SBLK_B, D), lambda s: (s, 0)),
            pl.BlockSpec((D, SBLK_B), lambda s: (0, s)),
            pl.BlockSpec((D, SBLK_B), lambda s: (0, s)),
        ],
        out_shape=[
            jax.ShapeDtypeStruct((S, D), jnp.bfloat16),
            jax.ShapeDtypeStruct((D, S), jnp.bfloat16),
            jax.ShapeDtypeStruct((D, S), jnp.bfloat16),
        ],
    )(W, x2.astype(jnp.bfloat16), init_val, gamma, beta)

    W2 = pl.pallas_call(
        _kD1,
        grid=(T // TBLK_D1,),
        in_specs=[
            pl.BlockSpec((TBLK_D1, D), lambda t: (t, 0)),
            pl.BlockSpec((TBLK_D1, D), lambda t: (t, 0)),
            pl.BlockSpec((D, S), lambda t: (0, 0)),
            pl.BlockSpec((D, S), lambda t: (0, 0)),
        ],
        out_specs=pl.BlockSpec((TBLK_D1, S), lambda t: (t, 0)),
        out_shape=jax.ShapeDtypeStruct((T, S), jnp.bfloat16),
    )(qh, ql, vth, vtl)

    y = pl.pallas_call(
        _kD2,
        grid=(T // TBLK_D2,),
        in_specs=[
            pl.BlockSpec((TBLK_D2, S), lambda t: (t, 0)),
            pl.BlockSpec((S, D), lambda t: (0, 0)),
            pl.BlockSpec((TBLK_D2, D), lambda t: (t, 0)),
            pl.BlockSpec((1, 1), lambda t: (0, 0)),
        ],
        out_specs=pl.BlockSpec((TBLK_D2, D), lambda t: (t, 0)),
        out_shape=jax.ShapeDtypeStruct((T, D), jnp.float32),
    )(W2, vh, x2, gate)

    return y.reshape(B, T, D)
